# Initial kernel scaffold; baseline (speedup 1.0000x reference)
import functools

import jax
import jax.numpy as jnp
from jax import lax
from jax.experimental import pallas as pl
from jax.experimental.pallas import tpu as pltpu
from jax.experimental.pallas import tpu_sc as plsc

N = 10000
E = 320000
D = 128
K = 6
OUT = 128

NC = 2
NS = 16
L = 16

H = D // NC
EW = E // NS
EC = 80
NCH_E = EW // EC
RC = 80
NCH_R = N // RC
RCH_PER_W = (NCH_R + NS - 1) // NS
NQ = H // L
SLOTS = 2 * K


def _bc16(v):
    return jnp.broadcast_to(jnp.asarray(v, jnp.int32), (L,))


def _iota16():
    return lax.iota(jnp.int32, L)


def _sc_body(xh2_ref, w3_ref, snd3_ref, rcv3_ref,
             tx_ref, sig_ref,
             acc_sh, deg_sh,
             sidx2, ridx2, wv2, dloc, rows, gidx, vbuf, pbuf, obuf, zbuf,
             sem):
    ci = lax.axis_index("c")
    si = lax.axis_index("s")
    iot = _iota16()
    zeros = jnp.zeros((L,), jnp.float32)

    for i in range(RCH_PER_W):
        ch = si + i * NS
        base = ch * RC

        @pl.when(ch < NCH_R)
        def _():
            row0 = ci * N + base
            pltpu.sync_copy(xh2_ref.at[pl.ds(row0, RC)], vbuf)
            pltpu.sync_copy(vbuf, tx_ref.at[pl.ds(row0, RC)])

    pltpu.sync_copy(snd3_ref.at[si], sidx2)
    pltpu.sync_copy(rcv3_ref.at[si], ridx2)
    pltpu.sync_copy(w3_ref.at[si], wv2)

    def zero_dloc(i, c):
        plsc.store_scatter(dloc, [iot + i * L], zeros)
        return c
    lax.fori_loop(0, N // L, zero_dloc, 0)
    for g in range(RC // L):
        plsc.store_scatter(zbuf, [iot + g * L], zeros)

    for i in range(RCH_PER_W):
        ch = si + i * NS
        base = ch * RC

        @pl.when(ch < NCH_R)
        def _():
            pltpu.sync_copy(zbuf, deg_sh.at[pl.ds(base, RC)])
    plsc.subcore_barrier()

    def deg_chunk(r, m2):
        for g in range(EC // L):
            idx = plsc.load_gather(sidx2, [_bc16(r), iot + g * L])
            wv = plsc.load_gather(wv2, [_bc16(r), iot + g * L])
            plsc.addupdate_scatter(dloc, [idx], wv)
            m2 = jnp.maximum(m2, -wv)
        return m2
    m2 = lax.fori_loop(0, NCH_E, deg_chunk,
                       jnp.full((L,), -jnp.inf, jnp.float32))

    pltpu.sync_copy(dloc, deg_sh, add=True)
    plsc.subcore_barrier()
    pltpu.sync_copy(deg_sh, dloc)

    def max_chunk(i, mv):
        dv = plsc.load_gather(dloc, [iot + i * L])
        return jnp.maximum(mv, dv)
    mv = lax.fori_loop(0, N // L, max_chunk, m2)
    mmax = jnp.max(mv)
    sigma = 1.0 / mmax
    gam2 = 0.5 * mmax * mmax
    gam3 = 0.25 * mmax * mmax

    @pl.when(jnp.logical_and(ci == 0, si == 0))
    def _():
        plsc.store_scatter(zbuf, [iot], jnp.broadcast_to(sigma, (L,)))
        pltpu.sync_copy(zbuf.at[pl.ds(0, L)], sig_ref)
        plsc.store_scatter(zbuf, [iot], zeros)

    def round_body(t, c):
        gam = jnp.where(t == 0, 0.0, jnp.where(t == 1, gam2, gam3))
        src_row = (2 * t + ci) * N
        prv_row = (2 * jnp.maximum(t - 1, 0) + ci) * N
        dst_row = (2 * (t + 1) + ci) * N

        for i in range(RCH_PER_W):
            ch = si + i * NS
            base = ch * RC

            @pl.when(ch < NCH_R)
            def _():
                pltpu.sync_copy(tx_ref.at[pl.ds(src_row + base, RC)], vbuf)
                pltpu.sync_copy(tx_ref.at[pl.ds(prv_row + base, RC)], pbuf)

                def row_body(r, c2):
                    dspl = plsc.load_gather(dloc, [_bc16(base + r)])
                    for q in range(NQ):
                        col = iot + q * L
                        vv = plsc.load_gather(vbuf, [_bc16(r), col])
                        pv = plsc.load_gather(pbuf, [_bc16(r), col])
                        plsc.store_scatter(obuf, [_bc16(r), col],
                                           dspl * vv - gam * pv)
                    return c2
                lax.fori_loop(0, RC, row_body, 0)
                pltpu.sync_copy(obuf, acc_sh.at[pl.ds(base, RC)])
        plsc.subcore_barrier()

        def edge_chunk(j, c2):
            for g in range(EC // L):
                rv = plsc.load_gather(ridx2, [_bc16(j), iot + g * L])
                plsc.store_scatter(gidx, [iot + g * L], rv + src_row)
            pltpu.async_copy(tx_ref.at[gidx], rows, sem).wait()

            def edge_body(e, c3):
                wspl = -plsc.load_gather(wv2, [_bc16(j), _bc16(e)])
                for q in range(NQ):
                    col = iot + q * L
                    rvv = plsc.load_gather(rows, [_bc16(e), col])
                    plsc.store_scatter(rows, [_bc16(e), col], rvv * wspl)
                return c3
            lax.fori_loop(0, EC, edge_body, 0)

            pltpu.sync_copy(rows, acc_sh.at[sidx2.at[j]], add=True)
            return c2
        lax.fori_loop(0, NCH_E, edge_chunk, 0)
        plsc.subcore_barrier()

        for i in range(RCH_PER_W):
            ch = si + i * NS
            base = ch * RC

            @pl.when(ch < NCH_R)
            def _():
                pltpu.sync_copy(acc_sh.at[pl.ds(base, RC)],
                                tx_ref.at[pl.ds(dst_row + base, RC)])
        plsc.subcore_barrier()
        return c

    lax.fori_loop(0, K - 1, round_body, 0)


@jax.jit
def _sc_cheb(xh2, w3, snd3, rcv3):
    mesh = plsc.VectorSubcoreMesh(core_axis_name="c", subcore_axis_name="s")
    return pl.kernel(
        _sc_body,
        out_type=(
            jax.ShapeDtypeStruct((SLOTS * N, H), jnp.float32),
            jax.ShapeDtypeStruct((L,), jnp.float32),
        ),
        mesh=mesh,
        scratch_types=[
            pltpu.VMEM_SHARED((N, H), jnp.float32),
            pltpu.VMEM_SHARED((N,), jnp.float32),
            pltpu.VMEM((NCH_E, EC), jnp.int32),
            pltpu.VMEM((NCH_E, EC), jnp.int32),
            pltpu.VMEM((NCH_E, EC), jnp.float32),
            pltpu.VMEM((N,), jnp.float32),
            pltpu.VMEM((EC, H), jnp.float32),
            pltpu.VMEM((EC,), jnp.int32),
            pltpu.VMEM((RC, H), jnp.float32),
            pltpu.VMEM((RC, H), jnp.float32),
            pltpu.VMEM((RC, H), jnp.float32),
            pltpu.VMEM((RC,), jnp.float32),
            pltpu.SemaphoreType.DMA,
        ],
    )(xh2, w3, snd3, rcv3)


RB = 1000
NRB = N // RB


def _tc_body(sig_ref, tx_ref, w_ref, bsum_ref, out_ref):
    s = pl.program_id(1)
    k = s // 2
    sg = sig_ref[0]
    sk = jnp.where(k == 0, 1.0, sg * (2.0 * sg) ** 0)
    p = 1.0
    for kk in range(1, K):
        p = p * (2.0 * sg) if kk > 1 else sg
        sk = jnp.where(k == kk, p, sk)
    contrib = jnp.dot(tx_ref[...], w_ref[0],
                      preferred_element_type=jnp.float32) * sk

    @pl.when(s == 0)
    def _():
        out_ref[...] = contrib + bsum_ref[...]

    @pl.when(s > 0)
    def _():
        out_ref[...] = out_ref[...] + contrib


@jax.jit
def _tc_combine(tx, sig, w3, bsum):
    grid = (NRB, SLOTS)
    return pl.pallas_call(
        _tc_body,
        grid=grid,
        in_specs=[
            pl.BlockSpec(memory_space=pltpu.SMEM),
            pl.BlockSpec((RB, H), lambda i, s: (s * NRB + i, 0)),
            pl.BlockSpec((1, H, OUT), lambda i, s: (s, 0, 0)),
            pl.BlockSpec((1, OUT), lambda i, s: (0, 0)),
        ],
        out_specs=pl.BlockSpec((RB, OUT), lambda i, s: (i, 0)),
        out_shape=jax.ShapeDtypeStruct((N, OUT), jnp.float32),
    )(sig, tx, w3, bsum)


def kernel(x, edge_weight, W, b, bias, senders, receivers):
    xh2 = x.reshape(N, NC, H).transpose(1, 0, 2).reshape(NC * N, H)
    w3 = edge_weight.reshape(NS, NCH_E, EC)
    snd3 = senders.astype(jnp.int32).reshape(NS, NCH_E, EC)
    rcv3 = receivers.astype(jnp.int32).reshape(NS, NCH_E, EC)
    tx, sig = _sc_cheb(xh2, w3, snd3, rcv3)
    w3d = W.reshape(K, NC, H, OUT).reshape(SLOTS, H, OUT)
    bsum = (b.sum(axis=0) + bias).reshape(1, OUT)
    return _tc_combine(tx, sig, w3d, bsum)


# speedup vs baseline: 1.8520x; 1.8520x over previous
import functools

import jax
import jax.numpy as jnp
from jax import lax
from jax.experimental import pallas as pl
from jax.experimental.pallas import tpu as pltpu
from jax.experimental.pallas import tpu_sc as plsc

N = 10000
E = 320000
D = 128
K = 6
OUT = 128

NC = 2
NS = 16
L = 16

H = D // NC
EW = E // NS
EC = 80
NCH_E = EW // EC
RC = 80
NCH_R = N // RC
RCH_PER_W = (NCH_R + NS - 1) // NS
NQ = H // L
SLOTS = 2 * K
DR = (N + L - 1) // L + 15
DZW = DR // NS


def _bc16(v):
    return jnp.broadcast_to(jnp.asarray(v, jnp.int32), (L,))


def _iota16():
    return lax.iota(jnp.int32, L)


def _sc_body(xh2_ref, w3_ref, snd3_ref, rcv_ref,
             tx_ref, sig_ref,
             acc_sh, deg_sh,
             sidx2, wv2, dloc, rows, gidx, rbuf, vbuf, pbuf, obuf,
             zrow, idbuf, sgbuf, sem):
    ci = lax.axis_index("c")
    si = lax.axis_index("s")
    iot = _iota16()
    zeros = jnp.zeros((L,), jnp.float32)

    for i in range(RCH_PER_W):
        ch = si + i * NS
        base = ch * RC

        @pl.when(ch < NCH_R)
        def _():
            row0 = ci * N + base
            pltpu.sync_copy(xh2_ref.at[pl.ds(row0, RC)], vbuf)
            pltpu.sync_copy(vbuf, tx_ref.at[pl.ds(row0, RC)])

    pltpu.sync_copy(snd3_ref.at[si], sidx2)
    pltpu.sync_copy(w3_ref.at[si], wv2)

    def zero_dloc(i, c):
        plsc.store_scatter(dloc, [_bc16(i), iot], zeros)
        return c
    lax.fori_loop(0, DR, zero_dloc, 0)
    for r in range(DZW):
        plsc.store_scatter(zrow, [_bc16(r), iot], zeros)

    def fill_id(i, c):
        plsc.store_scatter(idbuf, [iot + i * L], iot + i * L)
        return c
    lax.fori_loop(0, DR // L, fill_id, 0)

    pltpu.sync_copy(zrow, deg_sh.at[pl.ds(si * DZW, DZW)])
    plsc.subcore_barrier()

    def deg_chunk(r, m2):
        for g in range(EC // L):
            idx = plsc.load_gather(sidx2, [_bc16(r), iot + g * L])
            wv = plsc.load_gather(wv2, [_bc16(r), iot + g * L])
            plsc.addupdate_scatter(
                dloc,
                [lax.shift_right_logical(idx, 4), jnp.bitwise_and(idx, 15)],
                wv)
            m2 = jnp.maximum(m2, -wv)
        return m2
    m2 = lax.fori_loop(0, NCH_E, deg_chunk,
                       jnp.full((L,), -jnp.inf, jnp.float32))

    pltpu.sync_copy(dloc, deg_sh.at[idbuf], add=True)
    plsc.subcore_barrier()
    pltpu.sync_copy(deg_sh, dloc)

    def max_chunk(i, mv):
        dv = plsc.load_gather(dloc, [_bc16(i), iot])
        return jnp.maximum(mv, dv)
    mv = lax.fori_loop(0, DR, max_chunk, m2)
    mmax = jnp.max(mv)
    gam2 = 0.5 * mmax * mmax
    gam3 = 0.25 * mmax * mmax

    @pl.when(jnp.logical_and(ci == 0, si == 0))
    def _():
        sigv = jnp.full((L,), 1.0, jnp.float32) / jnp.broadcast_to(mmax, (L,))
        plsc.store_scatter(sgbuf, [iot], sigv)
        pltpu.sync_copy(sgbuf, sig_ref)

    def round_body(t, c):
        gam = jnp.where(t == 0, 0.0, jnp.where(t == 1, gam2, gam3))
        src_row = (2 * t + ci) * N
        prv_row = (2 * jnp.maximum(t - 1, 0) + ci) * N
        dst_row = (2 * (t + 1) + ci) * N

        for i in range(RCH_PER_W):
            ch = si + i * NS
            base = ch * RC

            @pl.when(ch < NCH_R)
            def _():
                pltpu.sync_copy(tx_ref.at[pl.ds(src_row + base, RC)], vbuf)
                pltpu.sync_copy(tx_ref.at[pl.ds(prv_row + base, RC)], pbuf)

                def row_body(r, c2):
                    n = base + r
                    dspl = plsc.load_gather(
                        dloc, [_bc16(n >> 4), _bc16(n & 15)])
                    for q in range(NQ):
                        col = iot + q * L
                        vv = plsc.load_gather(vbuf, [_bc16(r), col])
                        pv = plsc.load_gather(pbuf, [_bc16(r), col])
                        plsc.store_scatter(obuf, [_bc16(r), col],
                                           dspl * vv - gam * pv)
                    return c2
                lax.fori_loop(0, RC, row_body, 0)
                pltpu.sync_copy(obuf, acc_sh.at[pl.ds(base, RC)])
        plsc.subcore_barrier()

        def edge_chunk(j, c2):
            pltpu.sync_copy(rcv_ref.at[pl.ds(si * EW + j * EC, EC)], rbuf)
            for g in range(EC // L):
                rv = plsc.load_gather(rbuf, [iot + g * L])
                plsc.store_scatter(gidx, [iot + g * L], rv + src_row)
            pltpu.async_copy(tx_ref.at[gidx], rows, sem).wait()

            def edge_body(e, c3):
                wspl = -plsc.load_gather(wv2, [_bc16(j), _bc16(e)])
                for q in range(NQ):
                    col = iot + q * L
                    rvv = plsc.load_gather(rows, [_bc16(e), col])
                    plsc.store_scatter(rows, [_bc16(e), col], rvv * wspl)
                return c3
            lax.fori_loop(0, EC, edge_body, 0)

            pltpu.sync_copy(rows, acc_sh.at[sidx2.at[j]], add=True)
            return c2
        lax.fori_loop(0, NCH_E, edge_chunk, 0)
        plsc.subcore_barrier()

        for i in range(RCH_PER_W):
            ch = si + i * NS
            base = ch * RC

            @pl.when(ch < NCH_R)
            def _():
                pltpu.sync_copy(acc_sh.at[pl.ds(base, RC)],
                                tx_ref.at[pl.ds(dst_row + base, RC)])
        plsc.subcore_barrier()
        return c

    lax.fori_loop(0, K - 1, round_body, 0)


@jax.jit
def _sc_cheb(xh2, w3, snd3, rcv3):
    mesh = plsc.VectorSubcoreMesh(core_axis_name="c", subcore_axis_name="s",
                                  num_cores=NC, num_subcores=NS)
    return pl.kernel(
        _sc_body,
        out_type=(
            jax.ShapeDtypeStruct((SLOTS * N, H), jnp.float32),
            jax.ShapeDtypeStruct((L,), jnp.float32),
        ),
        mesh=mesh,
        compiler_params=pltpu.CompilerParams(needs_layout_passes=False,
                                             use_tc_tiling_on_sc=False),
        scratch_types=[
            pltpu.VMEM_SHARED((N, H), jnp.float32),
            pltpu.VMEM_SHARED((DR, L), jnp.float32),
            pltpu.VMEM((NCH_E, EC), jnp.int32),
            pltpu.VMEM((NCH_E, EC), jnp.float32),
            pltpu.VMEM((DR, L), jnp.float32),
            pltpu.VMEM((EC, H), jnp.float32),
            pltpu.VMEM((EC,), jnp.int32),
            pltpu.VMEM((EC,), jnp.int32),
            pltpu.VMEM((RC, H), jnp.float32),
            pltpu.VMEM((RC, H), jnp.float32),
            pltpu.VMEM((RC, H), jnp.float32),
            pltpu.VMEM((DZW, L), jnp.float32),
            pltpu.VMEM((DR,), jnp.int32),
            pltpu.VMEM((L,), jnp.float32),
            pltpu.SemaphoreType.DMA,
        ],
    )(xh2, w3, snd3, rcv3)


RB = 1000
NRB = N // RB


def _tc_body(sig_ref, tx_ref, w_ref, bsum_ref, out_ref):
    s = pl.program_id(1)
    k = s // 2
    sg = sig_ref[0]
    sk = jnp.float32(1.0)
    p = jnp.float32(1.0)
    for kk in range(1, K):
        p = p * sg if kk == 1 else p * (2.0 * sg)
        sk = jnp.where(k == kk, p, sk)
    contrib = jnp.dot(tx_ref[...], w_ref[0],
                      preferred_element_type=jnp.float32) * sk

    @pl.when(s == 0)
    def _():
        out_ref[...] = contrib + bsum_ref[...]

    @pl.when(s > 0)
    def _():
        out_ref[...] = out_ref[...] + contrib


@jax.jit
def _tc_combine(tx, sig, w3, bsum):
    grid = (NRB, SLOTS)
    return pl.pallas_call(
        _tc_body,
        grid=grid,
        in_specs=[
            pl.BlockSpec(memory_space=pltpu.SMEM),
            pl.BlockSpec((RB, H), lambda i, s: (s * NRB + i, 0)),
            pl.BlockSpec((1, H, OUT), lambda i, s: (s, 0, 0)),
            pl.BlockSpec((1, OUT), lambda i, s: (0, 0)),
        ],
        out_specs=pl.BlockSpec((RB, OUT), lambda i, s: (i, 0)),
        out_shape=jax.ShapeDtypeStruct((N, OUT), jnp.float32),
    )(sig, tx, w3, bsum)


def kernel(x, edge_weight, W, b, bias, senders, receivers):
    xh2 = x.reshape(N, NC, H).transpose(1, 0, 2).reshape(NC * N, H)
    w3 = edge_weight.reshape(NS, NCH_E, EC)
    snd3 = senders.astype(jnp.int32).reshape(NS, NCH_E, EC)
    rcv3 = receivers.astype(jnp.int32)
    tx, sig = _sc_cheb(xh2, w3, snd3, rcv3)
    w3d = W.reshape(K, NC, H, OUT).reshape(SLOTS, H, OUT)
    bsum = (b.sum(axis=0) + bias).reshape(1, OUT)
    return _tc_combine(tx, sig, w3d, bsum)


# speedup vs baseline: 2.4486x; 1.3221x over previous
import functools

import jax
import jax.numpy as jnp
from jax import lax
from jax.experimental import pallas as pl
from jax.experimental.pallas import tpu as pltpu
from jax.experimental.pallas import tpu_sc as plsc

N = 10000
E = 320000
D = 128
K = 6
OUT = 128

NC = 2
NS = 16
L = 16

H = D // NC
EW = E // NS
EC = 80
NCH_E = EW // EC
RC = 80
NCH_R = N // RC
RCH_PER_W = (NCH_R + NS - 1) // NS
NQ = H // L
SLOTS = 2 * K
DR = (N + L - 1) // L + 15
DZW = DR // NS


def _bc16(v):
    return jnp.broadcast_to(jnp.asarray(v, jnp.int32), (L,))


def _iota16():
    return lax.iota(jnp.int32, L)


def _sc_body(xh2_ref, w3_ref, snd3_ref, rcv_ref,
             tx_ref, sig_ref,
             acc_sh, deg_sh,
             sidx2, wv2, dloc, rows, rows1, gidx, gidx1, rbuf,
             vbuf, pbuf, obuf, zrow, idbuf, sgbuf,
             sem_g0, sem_g1, sem_s0, sem_s1, sem_r, sem_d):
    ci = lax.axis_index("c")
    si = lax.axis_index("s")
    iot = _iota16()
    zeros = jnp.zeros((L,), jnp.float32)

    for i in range(RCH_PER_W):
        ch = si + i * NS
        base = ch * RC

        @pl.when(ch < NCH_R)
        def _():
            row0 = ci * N + base
            pltpu.sync_copy(xh2_ref.at[pl.ds(row0, RC)], vbuf)
            pltpu.sync_copy(vbuf, tx_ref.at[pl.ds(row0, RC)])

    pltpu.sync_copy(snd3_ref.at[si], sidx2)
    pltpu.sync_copy(w3_ref.at[si], wv2)

    def zero_dloc(i, c):
        plsc.store_scatter(dloc, [_bc16(i), iot], zeros)
        return c
    lax.fori_loop(0, DR, zero_dloc, 0)
    for r in range(DZW):
        plsc.store_scatter(zrow, [_bc16(r), iot], zeros)

    def fill_id(i, c):
        plsc.store_scatter(idbuf, [iot + i * L], iot + i * L)
        return c
    lax.fori_loop(0, DR // L, fill_id, 0)

    pltpu.sync_copy(zrow, deg_sh.at[pl.ds(si * DZW, DZW)])
    plsc.subcore_barrier()

    def deg_chunk(r, m2):
        for g in range(EC // L):
            idx = plsc.load_gather(sidx2, [_bc16(r), iot + g * L])
            wv = plsc.load_gather(wv2, [_bc16(r), iot + g * L])
            plsc.addupdate_scatter(
                dloc,
                [lax.shift_right_logical(idx, 4), jnp.bitwise_and(idx, 15)],
                wv)
            m2 = jnp.maximum(m2, -wv)
        return m2
    m2 = lax.fori_loop(0, NCH_E, deg_chunk,
                       jnp.full((L,), -jnp.inf, jnp.float32))

    pltpu.sync_copy(dloc, deg_sh.at[idbuf], add=True)
    plsc.subcore_barrier()
    pltpu.sync_copy(deg_sh, dloc)

    def max_chunk(i, mv):
        dv = plsc.load_gather(dloc, [_bc16(i), iot])
        return jnp.maximum(mv, dv)
    mv = lax.fori_loop(0, DR, max_chunk, m2)
    mmax = jnp.max(mv)
    gam2 = 0.5 * mmax * mmax
    gam3 = 0.25 * mmax * mmax

    @pl.when(jnp.logical_and(ci == 0, si == 0))
    def _():
        sigv = jnp.full((L,), 1.0, jnp.float32) / jnp.broadcast_to(mmax, (L,))
        plsc.store_scatter(sgbuf, [iot], sigv)
        pltpu.sync_copy(sgbuf, sig_ref)

    def round_body(t, c):
        gam = jnp.where(t == 0, 0.0, jnp.where(t == 1, gam2, gam3))
        src_row = (2 * t + ci) * N
        prv_row = (2 * jnp.maximum(t - 1, 0) + ci) * N
        dst_row = (2 * (t + 1) + ci) * N

        for i in range(RCH_PER_W):
            ch = si + i * NS
            base = ch * RC

            @pl.when(ch < NCH_R)
            def _():
                pltpu.sync_copy(tx_ref.at[pl.ds(src_row + base, RC)], vbuf)
                pltpu.sync_copy(tx_ref.at[pl.ds(prv_row + base, RC)], pbuf)

                def row_body(r, c2):
                    n = base + r
                    dspl = plsc.load_gather(
                        dloc, [_bc16(n >> 4), _bc16(n & 15)])
                    for q in range(NQ):
                        col = iot + q * L
                        vv = plsc.load_gather(vbuf, [_bc16(r), col])
                        pv = plsc.load_gather(pbuf, [_bc16(r), col])
                        plsc.store_scatter(obuf, [_bc16(r), col],
                                           dspl * vv - gam * pv)
                    return c2
                lax.fori_loop(0, RC, row_body, 0)
                pltpu.sync_copy(obuf, acc_sh.at[pl.ds(base, RC)])
        plsc.subcore_barrier()

        rows_b = (rows, rows1)
        gidx_b = (gidx, gidx1)
        semg_b = (sem_g0, sem_g1)
        sems_b = (sem_s0, sem_s1)

        def _fetch_rbuf(j):
            pltpu.async_copy(rcv_ref.at[pl.ds(si * EW + j * EC, EC)],
                             rbuf, sem_r)

        def _wait_rbuf():
            pltpu.make_async_copy(rcv_ref.at[pl.ds(0, EC)], rbuf,
                                  sem_r).wait()

        def _build_gidx(b):
            for g in range(EC // L):
                rv = plsc.load_gather(rbuf, [iot + g * L])
                plsc.store_scatter(gidx_b[b], [iot + g * L], rv + src_row)

        def _start_gather(b):
            pltpu.async_copy(tx_ref.at[gidx_b[b]], rows_b[b], semg_b[b])

        def _wait_gather(b):
            pltpu.make_async_copy(tx_ref.at[pl.ds(0, EC)], rows_b[b],
                                  semg_b[b]).wait()

        def _compute(b, j):
            rw = rows_b[b]

            def eg(g5, c3):
                for k16 in range(L):
                    e = g5 * L + k16
                    wspl = -plsc.load_gather(wv2, [_bc16(j), _bc16(e)])
                    for q in range(NQ):
                        col = iot + q * L
                        rvv = plsc.load_gather(rw, [_bc16(e), col])
                        plsc.store_scatter(rw, [_bc16(e), col], rvv * wspl)
                return c3
            lax.fori_loop(0, EC // L, eg, 0)

        def _start_scatter(b, j):
            pltpu.async_copy(rows_b[b], acc_sh.at[sidx2.at[j]], sems_b[b],
                             add=True)

        def _wait_scatter(b):
            pltpu.make_async_copy(rows_b[b], acc_sh.at[pl.ds(0, EC)],
                                  sems_b[b]).wait()

        def _pair_body(j2, prefetch_next, wait_s1_first):
            j = 2 * j2
            jn = j + 1
            _fetch_rbuf(jn)
            _wait_gather(0)
            _compute(0, j)
            _wait_rbuf()
            _build_gidx(1)
            if wait_s1_first:
                _wait_scatter(1)
            _start_gather(1)
            _start_scatter(0, j)
            if prefetch_next:
                _fetch_rbuf(j + 2)
            _wait_gather(1)
            _compute(1, jn)
            if prefetch_next:
                _wait_rbuf()
                _build_gidx(0)
            _wait_scatter(0)
            if prefetch_next:
                _start_gather(0)
            _start_scatter(1, jn)

        pltpu.sync_copy(rcv_ref.at[pl.ds(si * EW, EC)], rbuf)
        _build_gidx(0)
        _start_gather(0)
        _pair_body(0, True, False)

        def pair_loop(j2, c2):
            _pair_body(j2, True, True)
            return c2
        lax.fori_loop(1, NCH_E // 2 - 1, pair_loop, 0)
        _pair_body(NCH_E // 2 - 1, False, True)
        _wait_scatter(1)
        plsc.subcore_barrier()

        for i in range(RCH_PER_W):
            ch = si + i * NS
            base = ch * RC

            @pl.when(ch < NCH_R)
            def _():
                pltpu.async_copy(acc_sh.at[pl.ds(base, RC)],
                                 tx_ref.at[pl.ds(dst_row + base, RC)],
                                 sem_d)
        for i in range(RCH_PER_W):
            ch = si + i * NS
            base = ch * RC

            @pl.when(ch < NCH_R)
            def _():
                pltpu.make_async_copy(
                    acc_sh.at[pl.ds(base, RC)],
                    tx_ref.at[pl.ds(dst_row + base, RC)],
                    sem_d).wait()
        plsc.subcore_barrier()
        return c

    lax.fori_loop(0, K - 1, round_body, 0)


@jax.jit
def _sc_cheb(xh2, w3, snd3, rcv3):
    mesh = plsc.VectorSubcoreMesh(core_axis_name="c", subcore_axis_name="s",
                                  num_cores=NC, num_subcores=NS)
    return pl.kernel(
        _sc_body,
        out_type=(
            jax.ShapeDtypeStruct((SLOTS * N, H), jnp.float32),
            jax.ShapeDtypeStruct((L,), jnp.float32),
        ),
        mesh=mesh,
        compiler_params=pltpu.CompilerParams(needs_layout_passes=False,
                                             use_tc_tiling_on_sc=False),
        scratch_types=[
            pltpu.VMEM_SHARED((N, H), jnp.float32),
            pltpu.VMEM_SHARED((DR, L), jnp.float32),
            pltpu.VMEM((NCH_E, EC), jnp.int32),
            pltpu.VMEM((NCH_E, EC), jnp.float32),
            pltpu.VMEM((DR, L), jnp.float32),
            pltpu.VMEM((EC, H), jnp.float32),
            pltpu.VMEM((EC, H), jnp.float32),
            pltpu.VMEM((EC,), jnp.int32),
            pltpu.VMEM((EC,), jnp.int32),
            pltpu.VMEM((EC,), jnp.int32),
            pltpu.VMEM((RC, H), jnp.float32),
            pltpu.VMEM((RC, H), jnp.float32),
            pltpu.VMEM((RC, H), jnp.float32),
            pltpu.VMEM((DZW, L), jnp.float32),
            pltpu.VMEM((DR,), jnp.int32),
            pltpu.VMEM((L,), jnp.float32),
            pltpu.SemaphoreType.DMA,
            pltpu.SemaphoreType.DMA,
            pltpu.SemaphoreType.DMA,
            pltpu.SemaphoreType.DMA,
            pltpu.SemaphoreType.DMA,
            pltpu.SemaphoreType.DMA,
        ],
    )(xh2, w3, snd3, rcv3)


RB = 1000
NRB = N // RB


def _tc_body(sig_ref, tx_ref, w_ref, bsum_ref, out_ref):
    s = pl.program_id(1)
    k = s // 2
    sg = sig_ref[0]
    sk = jnp.float32(1.0)
    p = jnp.float32(1.0)
    for kk in range(1, K):
        p = p * sg if kk == 1 else p * (2.0 * sg)
        sk = jnp.where(k == kk, p, sk)
    contrib = jnp.dot(tx_ref[...], w_ref[0],
                      preferred_element_type=jnp.float32) * sk

    @pl.when(s == 0)
    def _():
        out_ref[...] = contrib + bsum_ref[...]

    @pl.when(s > 0)
    def _():
        out_ref[...] = out_ref[...] + contrib


@jax.jit
def _tc_combine(tx, sig, w3, bsum):
    grid = (NRB, SLOTS)
    return pl.pallas_call(
        _tc_body,
        grid=grid,
        in_specs=[
            pl.BlockSpec(memory_space=pltpu.SMEM),
            pl.BlockSpec((RB, H), lambda i, s: (s * NRB + i, 0)),
            pl.BlockSpec((1, H, OUT), lambda i, s: (s, 0, 0)),
            pl.BlockSpec((1, OUT), lambda i, s: (0, 0)),
        ],
        out_specs=pl.BlockSpec((RB, OUT), lambda i, s: (i, 0)),
        out_shape=jax.ShapeDtypeStruct((N, OUT), jnp.float32),
    )(sig, tx, w3, bsum)


def kernel(x, edge_weight, W, b, bias, senders, receivers):
    xh2 = x.reshape(N, NC, H).transpose(1, 0, 2).reshape(NC * N, H)
    w3 = edge_weight.reshape(NS, NCH_E, EC)
    snd3 = senders.astype(jnp.int32).reshape(NS, NCH_E, EC)
    rcv3 = receivers.astype(jnp.int32)
    tx, sig = _sc_cheb(xh2, w3, snd3, rcv3)
    w3d = W.reshape(K, NC, H, OUT).reshape(SLOTS, H, OUT)
    bsum = (b.sum(axis=0) + bias).reshape(1, OUT)
    return _tc_combine(tx, sig, w3d, bsum)


# speedup vs baseline: 4.8884x; 1.9964x over previous
import functools

import jax
import jax.numpy as jnp
from jax import lax
from jax.experimental import pallas as pl
from jax.experimental.pallas import tpu as pltpu
from jax.experimental.pallas import tpu_sc as plsc

N = 10000
E = 320000
D = 128
K = 6
OUT = 128

NC = 2
NS = 16
L = 16

H = D // NC
EW = E // NS
EC = 80
NCH_E = EW // EC
RC = 80
NCH_R = N // RC
RCH_PER_W = (NCH_R + NS - 1) // NS
NQ = H // L
SLOTS = 2 * K
DR = (N + L - 1) // L + 15
DZW = DR // NS


def _bc16(v):
    return jnp.broadcast_to(jnp.asarray(v, jnp.int32), (L,))


def _iota16():
    return lax.iota(jnp.int32, L)


def _sc_body(xh2_ref, w3_ref, snd3_ref, rcv_ref,
             tx_ref, sig_ref,
             acc_sh, deg_sh,
             sidx2, wv2, dloc, rows, rows1, gidx, gidx1, rbuf,
             vbuf, pbuf, obuf, zrow, idbuf, sgbuf,
             sem_g0, sem_g1, sem_s0, sem_s1, sem_r, sem_d):
    ci = lax.axis_index("c")
    si = lax.axis_index("s")
    iot = _iota16()
    zeros = jnp.zeros((L,), jnp.float32)

    for i in range(RCH_PER_W):
        ch = si + i * NS
        base = ch * RC

        @pl.when(ch < NCH_R)
        def _():
            row0 = ci * N + base
            pltpu.sync_copy(xh2_ref.at[pl.ds(row0, RC)], vbuf)
            pltpu.sync_copy(vbuf, tx_ref.at[pl.ds(row0, RC)])

    pltpu.sync_copy(snd3_ref.at[si], sidx2)
    pltpu.sync_copy(w3_ref.at[si], wv2)

    def zero_dloc(i, c):
        plsc.store_scatter(dloc, [_bc16(i), iot], zeros)
        return c
    lax.fori_loop(0, DR, zero_dloc, 0)
    for r in range(DZW):
        plsc.store_scatter(zrow, [_bc16(r), iot], zeros)

    def fill_id(i, c):
        plsc.store_scatter(idbuf, [iot + i * L], iot + i * L)
        return c
    lax.fori_loop(0, DR // L, fill_id, 0)

    pltpu.sync_copy(zrow, deg_sh.at[pl.ds(si * DZW, DZW)])
    plsc.subcore_barrier()

    def deg_chunk(r, m2):
        for g in range(EC // L):
            idx = plsc.load_gather(sidx2, [_bc16(r), iot + g * L])
            wv = plsc.load_gather(wv2, [_bc16(r), iot + g * L])
            plsc.addupdate_scatter(
                dloc,
                [lax.shift_right_logical(idx, 4), jnp.bitwise_and(idx, 15)],
                wv)
            m2 = jnp.maximum(m2, -wv)
        return m2
    m2 = lax.fori_loop(0, NCH_E, deg_chunk,
                       jnp.full((L,), -jnp.inf, jnp.float32))

    pltpu.sync_copy(dloc, deg_sh.at[idbuf], add=True)
    plsc.subcore_barrier()
    pltpu.sync_copy(deg_sh, dloc)

    def max_chunk(i, mv):
        dv = plsc.load_gather(dloc, [_bc16(i), iot])
        return jnp.maximum(mv, dv)
    mv = lax.fori_loop(0, DR, max_chunk, m2)
    mmax = jnp.max(mv)
    gam2 = 0.5 * mmax * mmax
    gam3 = 0.25 * mmax * mmax

    @pl.when(jnp.logical_and(ci == 0, si == 0))
    def _():
        sigv = jnp.full((L,), 1.0, jnp.float32) / jnp.broadcast_to(mmax, (L,))
        plsc.store_scatter(sgbuf, [iot], sigv)
        pltpu.sync_copy(sgbuf, sig_ref)

    def round_body(t, c):
        gam = jnp.where(t == 0, 0.0, jnp.where(t == 1, gam2, gam3))
        src_row = (2 * t + ci) * N
        prv_row = (2 * jnp.maximum(t - 1, 0) + ci) * N
        dst_row = (2 * (t + 1) + ci) * N

        for i in range(RCH_PER_W):
            ch = si + i * NS
            base = ch * RC

            @pl.when(ch < NCH_R)
            def _():
                pltpu.sync_copy(tx_ref.at[pl.ds(src_row + base, RC)], vbuf)
                pltpu.sync_copy(tx_ref.at[pl.ds(prv_row + base, RC)], pbuf)

                def row_body(r, c2):
                    n = base + r
                    dspl = plsc.load_gather(
                        dloc, [_bc16(n >> 4), _bc16(n & 15)])
                    for q in range(NQ):
                        col = iot + q * L
                        vv = plsc.load_gather(vbuf, [_bc16(r), col])
                        pv = plsc.load_gather(pbuf, [_bc16(r), col])
                        plsc.store_scatter(obuf, [_bc16(r), col],
                                           dspl * vv - gam * pv)
                    return c2
                lax.fori_loop(0, RC, row_body, 0)
                pltpu.sync_copy(obuf, acc_sh.at[pl.ds(base, RC)])
        plsc.subcore_barrier()

        rows_b = (rows, rows1)
        gidx_b = (gidx, gidx1)
        semg_b = (sem_g0, sem_g1)
        sems_b = (sem_s0, sem_s1)

        def _fetch_rbuf(j):
            pltpu.async_copy(rcv_ref.at[pl.ds(si * EW + j * EC, EC)],
                             rbuf, sem_r)

        def _wait_rbuf():
            pltpu.make_async_copy(rcv_ref.at[pl.ds(0, EC)], rbuf,
                                  sem_r).wait()

        def _build_gidx(b):
            for g in range(EC // L):
                sl = pl.ds(g * L, L)
                gidx_b[b][sl] = rbuf[sl] + src_row

        def _start_gather(b):
            pltpu.async_copy(tx_ref.at[gidx_b[b]], rows_b[b], semg_b[b])

        def _wait_gather(b):
            pltpu.make_async_copy(tx_ref.at[pl.ds(0, EC)], rows_b[b],
                                  semg_b[b]).wait()

        def _compute(b, j):
            rw = rows_b[b]
            for g in range(EC // L):
                wg = -wv2[j, pl.ds(g * L, L)]
                for k16 in range(L):
                    e = g * L + k16
                    w = jnp.broadcast_to(wg[k16], (L,))
                    for q in range(NQ):
                        sl = pl.ds(q * L, L)
                        rw[e, sl] = rw[e, sl] * w

        def _start_scatter(b, j):
            pltpu.async_copy(rows_b[b], acc_sh.at[sidx2.at[j]], sems_b[b],
                             add=True)

        def _wait_scatter(b):
            pltpu.make_async_copy(rows_b[b], acc_sh.at[pl.ds(0, EC)],
                                  sems_b[b]).wait()

        def _pair_body(j2, prefetch_next, wait_s1_first):
            j = 2 * j2
            jn = j + 1
            _fetch_rbuf(jn)
            _wait_gather(0)
            _compute(0, j)
            _wait_rbuf()
            _build_gidx(1)
            if wait_s1_first:
                _wait_scatter(1)
            _start_gather(1)
            _start_scatter(0, j)
            if prefetch_next:
                _fetch_rbuf(j + 2)
            _wait_gather(1)
            _compute(1, jn)
            if prefetch_next:
                _wait_rbuf()
                _build_gidx(0)
            _wait_scatter(0)
            if prefetch_next:
                _start_gather(0)
            _start_scatter(1, jn)

        pltpu.sync_copy(rcv_ref.at[pl.ds(si * EW, EC)], rbuf)
        _build_gidx(0)
        _start_gather(0)
        _pair_body(0, True, False)

        def pair_loop(j2, c2):
            _pair_body(j2, True, True)
            return c2
        lax.fori_loop(1, NCH_E // 2 - 1, pair_loop, 0)
        _pair_body(NCH_E // 2 - 1, False, True)
        _wait_scatter(1)
        plsc.subcore_barrier()

        for i in range(RCH_PER_W):
            ch = si + i * NS
            base = ch * RC

            @pl.when(ch < NCH_R)
            def _():
                pltpu.async_copy(acc_sh.at[pl.ds(base, RC)],
                                 tx_ref.at[pl.ds(dst_row + base, RC)],
                                 sem_d)
        for i in range(RCH_PER_W):
            ch = si + i * NS
            base = ch * RC

            @pl.when(ch < NCH_R)
            def _():
                pltpu.make_async_copy(
                    acc_sh.at[pl.ds(base, RC)],
                    tx_ref.at[pl.ds(dst_row + base, RC)],
                    sem_d).wait()
        plsc.subcore_barrier()
        return c

    lax.fori_loop(0, K - 1, round_body, 0)


@jax.jit
def _sc_cheb(xh2, w3, snd3, rcv3):
    mesh = plsc.VectorSubcoreMesh(core_axis_name="c", subcore_axis_name="s",
                                  num_cores=NC, num_subcores=NS)
    return pl.kernel(
        _sc_body,
        out_type=(
            jax.ShapeDtypeStruct((SLOTS * N, H), jnp.float32),
            jax.ShapeDtypeStruct((L,), jnp.float32),
        ),
        mesh=mesh,
        compiler_params=pltpu.CompilerParams(needs_layout_passes=False,
                                             use_tc_tiling_on_sc=False),
        scratch_types=[
            pltpu.VMEM_SHARED((N, H), jnp.float32),
            pltpu.VMEM_SHARED((DR, L), jnp.float32),
            pltpu.VMEM((NCH_E, EC), jnp.int32),
            pltpu.VMEM((NCH_E, EC), jnp.float32),
            pltpu.VMEM((DR, L), jnp.float32),
            pltpu.VMEM((EC, H), jnp.float32),
            pltpu.VMEM((EC, H), jnp.float32),
            pltpu.VMEM((EC,), jnp.int32),
            pltpu.VMEM((EC,), jnp.int32),
            pltpu.VMEM((EC,), jnp.int32),
            pltpu.VMEM((RC, H), jnp.float32),
            pltpu.VMEM((RC, H), jnp.float32),
            pltpu.VMEM((RC, H), jnp.float32),
            pltpu.VMEM((DZW, L), jnp.float32),
            pltpu.VMEM((DR,), jnp.int32),
            pltpu.VMEM((L,), jnp.float32),
            pltpu.SemaphoreType.DMA,
            pltpu.SemaphoreType.DMA,
            pltpu.SemaphoreType.DMA,
            pltpu.SemaphoreType.DMA,
            pltpu.SemaphoreType.DMA,
            pltpu.SemaphoreType.DMA,
        ],
    )(xh2, w3, snd3, rcv3)


RB = 1000
NRB = N // RB


def _tc_body(sig_ref, tx_ref, w_ref, bsum_ref, out_ref):
    s = pl.program_id(1)
    k = s // 2
    sg = sig_ref[0]
    sk = jnp.float32(1.0)
    p = jnp.float32(1.0)
    for kk in range(1, K):
        p = p * sg if kk == 1 else p * (2.0 * sg)
        sk = jnp.where(k == kk, p, sk)
    contrib = jnp.dot(tx_ref[...], w_ref[0],
                      preferred_element_type=jnp.float32) * sk

    @pl.when(s == 0)
    def _():
        out_ref[...] = contrib + bsum_ref[...]

    @pl.when(s > 0)
    def _():
        out_ref[...] = out_ref[...] + contrib


@jax.jit
def _tc_combine(tx, sig, w3, bsum):
    grid = (NRB, SLOTS)
    return pl.pallas_call(
        _tc_body,
        grid=grid,
        in_specs=[
            pl.BlockSpec(memory_space=pltpu.SMEM),
            pl.BlockSpec((RB, H), lambda i, s: (s * NRB + i, 0)),
            pl.BlockSpec((1, H, OUT), lambda i, s: (s, 0, 0)),
            pl.BlockSpec((1, OUT), lambda i, s: (0, 0)),
        ],
        out_specs=pl.BlockSpec((RB, OUT), lambda i, s: (i, 0)),
        out_shape=jax.ShapeDtypeStruct((N, OUT), jnp.float32),
    )(sig, tx, w3, bsum)


def kernel(x, edge_weight, W, b, bias, senders, receivers):
    xh2 = x.reshape(N, NC, H).transpose(1, 0, 2).reshape(NC * N, H)
    w3 = edge_weight.reshape(NS, NCH_E, EC)
    snd3 = senders.astype(jnp.int32).reshape(NS, NCH_E, EC)
    rcv3 = receivers.astype(jnp.int32)
    tx, sig = _sc_cheb(xh2, w3, snd3, rcv3)
    w3d = W.reshape(K, NC, H, OUT).reshape(SLOTS, H, OUT)
    bsum = (b.sum(axis=0) + bias).reshape(1, OUT)
    return _tc_combine(tx, sig, w3d, bsum)


# speedup vs baseline: 8.4104x; 1.7205x over previous
import functools

import jax
import jax.numpy as jnp
from jax import lax
from jax.experimental import pallas as pl
from jax.experimental.pallas import tpu as pltpu
from jax.experimental.pallas import tpu_sc as plsc

N = 10000
E = 320000
D = 128
K = 6
OUT = 128

NC = 2
NS = 16
L = 16

H = D // NC
EW = E // NS
EC = 80
EC2 = 2 * EC
NSC = EW // EC2
NCH_E = EW // EC
RC = 80
NCH_R = N // RC
RCH_PER_W = (NCH_R + NS - 1) // NS
NQ = H // L
SLOTS = 2 * K
DR = 640
DZW = DR // NS


def _bc16(v):
    return jnp.broadcast_to(jnp.asarray(v, jnp.int32), (L,))


def _iota16():
    return lax.iota(jnp.int32, L)


def _sc_body(xh2_ref, wf_ref, snd3_ref, rcv3_ref,
             tx_ref, sig_ref,
             acc_sh, deg_sh,
             ridx_all, dloc, rows0, rows1, rows2,
             sbuf0, sbuf1, sbuf2, wbuf0, wbuf1, wbuf2,
             gix0, gix1, gix2, vbuf, pbuf, obuf, zrow, idbuf, sgbuf,
             sem_g0, sem_g1, sem_g2, sem_s0, sem_s1, sem_s2,
             sem_f0, sem_f1, sem_f2, sem_d):
    ci = lax.axis_index("c")
    si = lax.axis_index("s")
    iot = _iota16()
    zeros = jnp.zeros((L,), jnp.float32)
    rows_b = (rows0, rows1, rows2)
    sbuf_b = (sbuf0, sbuf1, sbuf2)
    wbuf_b = (wbuf0, wbuf1, wbuf2)
    gix_b = (gix0, gix1, gix2)
    semg_b = (sem_g0, sem_g1, sem_g2)
    sems_b = (sem_s0, sem_s1, sem_s2)
    semf_b = (sem_f0, sem_f1, sem_f2)

    for i in range(RCH_PER_W):
        ch = si + i * NS
        base = ch * RC

        @pl.when(ch < NCH_R)
        def _():
            row0 = ci * N + base
            pltpu.sync_copy(xh2_ref.at[pl.ds(row0, RC)], vbuf)
            pltpu.sync_copy(vbuf, tx_ref.at[pl.ds(row0, RC)])
            pltpu.sync_copy(vbuf, acc_sh.at[pl.ds(base, RC)])

    pltpu.sync_copy(snd3_ref.at[si], ridx_all)

    def zero_dloc(i, c):
        plsc.store_scatter(dloc, [_bc16(i), iot], zeros)
        return c
    lax.fori_loop(0, DR, zero_dloc, 0)
    for r in range(DZW):
        plsc.store_scatter(zrow, [_bc16(r), iot], zeros)

    def fill_id(i, c):
        plsc.store_scatter(idbuf, [iot + i * L], iot + i * L)
        return c
    lax.fori_loop(0, DR // L, fill_id, 0)

    pltpu.sync_copy(zrow, deg_sh.at[pl.ds(si * DZW, DZW)])
    plsc.subcore_barrier()

    def deg_sc(jj, m2):
        pltpu.sync_copy(wf_ref.at[pl.ds(si * EW + jj * EC2, EC2)], wbuf0)
        for k in range(2):
            for g in range(EC // L):
                idx = ridx_all[2 * jj + k, pl.ds(g * L, L)]
                wv = wbuf0[pl.ds((k * (EC // L) + g) * L, L)]
                plsc.addupdate_scatter(
                    dloc,
                    [lax.shift_right_logical(idx, 4),
                     jnp.bitwise_and(idx, 15)],
                    wv)
                m2 = jnp.maximum(m2, -wv)
        return m2
    m2 = lax.fori_loop(0, NSC, deg_sc,
                       jnp.full((L,), -jnp.inf, jnp.float32))

    pltpu.sync_copy(dloc, deg_sh.at[idbuf], add=True)
    plsc.subcore_barrier()
    pltpu.sync_copy(deg_sh, dloc)

    def max_chunk(i, mv):
        return jnp.maximum(mv, dloc[i, pl.ds(0, L)])
    mv = lax.fori_loop(0, DR, max_chunk, m2)
    mmax = jnp.max(mv)
    gam2 = 0.5 * mmax * mmax
    gam3 = 0.25 * mmax * mmax

    @pl.when(jnp.logical_and(ci == 0, si == 0))
    def _():
        sigv = jnp.full((L,), 1.0, jnp.float32) / jnp.broadcast_to(mmax, (L,))
        plsc.store_scatter(sgbuf, [iot], sigv)
        pltpu.sync_copy(sgbuf, sig_ref)

    pltpu.sync_copy(rcv3_ref.at[si], ridx_all)

    def round_body(t, c):
        gam = jnp.where(t == 0, 0.0, jnp.where(t == 1, gam2, gam3))
        gamv = jnp.broadcast_to(gam, (L,))
        src_row = (2 * t + ci) * N
        prv_row = (2 * jnp.maximum(t - 1, 0) + ci) * N
        dst_row = (2 * (t + 1) + ci) * N

        def init_chunk(i, c2):
            ch = si + i * NS
            base = ch * RC

            @pl.when(ch < NCH_R)
            def _():
                pltpu.sync_copy(acc_sh.at[pl.ds(base, RC)], vbuf)
                pltpu.sync_copy(tx_ref.at[pl.ds(prv_row + base, RC)], pbuf)

                def grp_body(r5, c3):
                    dgrp = dloc[ch * (RC // L) + r5, pl.ds(0, L)]
                    for k16 in range(L):
                        r = r5 * L + k16
                        dspl = jnp.broadcast_to(dgrp[k16], (L,))
                        for q in range(NQ):
                            sl = pl.ds(q * L, L)
                            obuf[r, sl] = (dspl * vbuf[r, sl]
                                           - gamv * pbuf[r, sl])
                    return c3
                lax.fori_loop(0, RC // L, grp_body, 0)
                pltpu.sync_copy(obuf, acc_sh.at[pl.ds(base, RC)])
            return c2
        lax.fori_loop(0, RCH_PER_W, init_chunk, 0)
        plsc.subcore_barrier()

        def _build_gix(b, jj):
            for k in range(2):
                for g in range(EC // L):
                    sl = pl.ds(g * L, L)
                    gix_b[b][k, sl] = ridx_all[2 * jj + k, sl] + src_row

        def _start_gather(b):
            for k in range(2):
                pltpu.async_copy(tx_ref.at[gix_b[b].at[k]],
                                 rows_b[b].at[pl.ds(k * EC, EC)],
                                 semg_b[b])

        def _wait_gather(b):
            for k in range(2):
                pltpu.make_async_copy(
                    tx_ref.at[pl.ds(0, EC)],
                    rows_b[b].at[pl.ds(k * EC, EC)], semg_b[b]).wait()

        def _fetch_sw(b, jj):
            pltpu.async_copy(snd3_ref.at[si].at[pl.ds(2 * jj, 2)],
                             sbuf_b[b], semf_b[b])
            pltpu.async_copy(wf_ref.at[pl.ds(si * EW + jj * EC2, EC2)],
                             wbuf_b[b], semf_b[b])

        def _wait_sw(b):
            pltpu.make_async_copy(snd3_ref.at[si].at[pl.ds(0, 2)],
                                  sbuf_b[b], semf_b[b]).wait()
            pltpu.make_async_copy(wf_ref.at[pl.ds(0, EC2)],
                                  wbuf_b[b], semf_b[b]).wait()

        def _compute(b):
            rw = rows_b[b]
            wb = wbuf_b[b]

            def grp(g, c3):
                wg = -wb[pl.ds(g * L, L)]
                for k16 in range(L):
                    e = g * L + k16
                    w = jnp.broadcast_to(wg[k16], (L,))
                    for q in range(NQ):
                        sl = pl.ds(q * L, L)
                        rw[e, sl] = rw[e, sl] * w
                return c3
            lax.fori_loop(0, EC2 // L, grp, 0)

        def _start_scatter(b):
            for k in range(2):
                pltpu.async_copy(rows_b[b].at[pl.ds(k * EC, EC)],
                                 acc_sh.at[sbuf_b[b].at[k]],
                                 sems_b[b], add=True)

        def _wait_scatter(b):
            for k in range(2):
                pltpu.make_async_copy(
                    rows_b[b].at[pl.ds(k * EC, EC)],
                    acc_sh.at[pl.ds(0, EC)], sems_b[b]).wait()

        def _step(jj, b, wait_prev2, fetch_next):
            n1 = (b + 1) % 3
            if fetch_next:
                _build_gix(n1, jj + 1)
            if wait_prev2:
                _wait_scatter(n1)
            if fetch_next:
                _start_gather(n1)
                _fetch_sw(n1, jj + 1)
            _wait_gather(b)
            _wait_sw(b)
            _compute(b)
            _start_scatter(b)

        _fetch_sw(0, 0)
        _build_gix(0, 0)
        _start_gather(0)
        _step(0, 0, False, True)
        _step(1, 1, False, True)

        def tri_loop(p, c2):
            _step(3 * p + 2, 2, True, True)
            _step(3 * p + 3, 0, True, True)
            _step(3 * p + 4, 1, True, True)
            return c2
        lax.fori_loop(0, (NSC - 5) // 3, tri_loop, 0)
        _step(NSC - 3, 2, True, True)
        _step(NSC - 2, 0, True, True)
        _step(NSC - 1, 1, True, False)
        _wait_scatter(0)
        _wait_scatter(1)
        plsc.subcore_barrier()

        for i in range(RCH_PER_W):
            ch = si + i * NS
            base = ch * RC

            @pl.when(ch < NCH_R)
            def _():
                pltpu.async_copy(acc_sh.at[pl.ds(base, RC)],
                                 tx_ref.at[pl.ds(dst_row + base, RC)],
                                 sem_d)
        for i in range(RCH_PER_W):
            ch = si + i * NS
            base = ch * RC

            @pl.when(ch < NCH_R)
            def _():
                pltpu.make_async_copy(
                    acc_sh.at[pl.ds(base, RC)],
                    tx_ref.at[pl.ds(dst_row + base, RC)],
                    sem_d).wait()
        plsc.subcore_barrier()
        return c

    lax.fori_loop(0, K - 1, round_body, 0)


@jax.jit
def _sc_cheb(xh2, wf, snd3, rcv3):
    mesh = plsc.VectorSubcoreMesh(core_axis_name="c", subcore_axis_name="s",
                                  num_cores=NC, num_subcores=NS)
    return pl.kernel(
        _sc_body,
        out_type=(
            jax.ShapeDtypeStruct((SLOTS * N, H), jnp.float32),
            jax.ShapeDtypeStruct((L,), jnp.float32),
        ),
        mesh=mesh,
        compiler_params=pltpu.CompilerParams(needs_layout_passes=False,
                                             use_tc_tiling_on_sc=False),
        scratch_types=[
            pltpu.VMEM_SHARED((N, H), jnp.float32),
            pltpu.VMEM_SHARED((DR, L), jnp.float32),
            pltpu.VMEM((NCH_E, EC), jnp.int32),
            pltpu.VMEM((DR, L), jnp.float32),
            pltpu.VMEM((EC2, H), jnp.float32),
            pltpu.VMEM((EC2, H), jnp.float32),
            pltpu.VMEM((EC2, H), jnp.float32),
            pltpu.VMEM((2, EC), jnp.int32),
            pltpu.VMEM((2, EC), jnp.int32),
            pltpu.VMEM((2, EC), jnp.int32),
            pltpu.VMEM((EC2,), jnp.float32),
            pltpu.VMEM((EC2,), jnp.float32),
            pltpu.VMEM((EC2,), jnp.float32),
            pltpu.VMEM((2, EC), jnp.int32),
            pltpu.VMEM((2, EC), jnp.int32),
            pltpu.VMEM((2, EC), jnp.int32),
            pltpu.VMEM((RC, H), jnp.float32),
            pltpu.VMEM((RC, H), jnp.float32),
            pltpu.VMEM((RC, H), jnp.float32),
            pltpu.VMEM((DZW, L), jnp.float32),
            pltpu.VMEM((DR,), jnp.int32),
            pltpu.VMEM((L,), jnp.float32),
            pltpu.SemaphoreType.DMA,
            pltpu.SemaphoreType.DMA,
            pltpu.SemaphoreType.DMA,
            pltpu.SemaphoreType.DMA,
            pltpu.SemaphoreType.DMA,
            pltpu.SemaphoreType.DMA,
            pltpu.SemaphoreType.DMA,
            pltpu.SemaphoreType.DMA,
            pltpu.SemaphoreType.DMA,
            pltpu.SemaphoreType.DMA,
        ],
    )(xh2, wf, snd3, rcv3)


RB = 1000
NRB = N // RB


def _tc_body(sig_ref, tx_ref, w_ref, bsum_ref, out_ref):
    s = pl.program_id(1)
    k = s // 2
    sg = sig_ref[0]
    sk = jnp.float32(1.0)
    p = jnp.float32(1.0)
    for kk in range(1, K):
        p = p * sg if kk == 1 else p * (2.0 * sg)
        sk = jnp.where(k == kk, p, sk)
    contrib = jnp.dot(tx_ref[...], w_ref[0],
                      preferred_element_type=jnp.float32) * sk

    @pl.when(s == 0)
    def _():
        out_ref[...] = contrib + bsum_ref[...]

    @pl.when(s > 0)
    def _():
        out_ref[...] = out_ref[...] + contrib


@jax.jit
def _tc_combine(tx, sig, w3, bsum):
    grid = (NRB, SLOTS)
    return pl.pallas_call(
        _tc_body,
        grid=grid,
        in_specs=[
            pl.BlockSpec(memory_space=pltpu.SMEM),
            pl.BlockSpec((RB, H), lambda i, s: (s * NRB + i, 0)),
            pl.BlockSpec((1, H, OUT), lambda i, s: (s, 0, 0)),
            pl.BlockSpec((1, OUT), lambda i, s: (0, 0)),
        ],
        out_specs=pl.BlockSpec((RB, OUT), lambda i, s: (i, 0)),
        out_shape=jax.ShapeDtypeStruct((N, OUT), jnp.float32),
    )(sig, tx, w3, bsum)


def kernel(x, edge_weight, W, b, bias, senders, receivers):
    xh2 = x.reshape(N, NC, H).transpose(1, 0, 2).reshape(NC * N, H)
    wf = edge_weight
    snd3 = senders.astype(jnp.int32).reshape(NS, NCH_E, EC)
    rcv3 = receivers.astype(jnp.int32).reshape(NS, NCH_E, EC)
    tx, sig = _sc_cheb(xh2, wf, snd3, rcv3)
    w3d = W.reshape(K, NC, H, OUT).reshape(SLOTS, H, OUT)
    bsum = (b.sum(axis=0) + bias).reshape(1, OUT)
    return _tc_combine(tx, sig, w3d, bsum)


# speedup vs baseline: 8.7451x; 1.0398x over previous
import functools

import jax
import jax.numpy as jnp
from jax import lax
from jax.experimental import pallas as pl
from jax.experimental.pallas import tpu as pltpu
from jax.experimental.pallas import tpu_sc as plsc

N = 10000
E = 320000
D = 128
K = 6
OUT = 128

NC = 2
NS = 16
L = 16

H = D // NC
EW = E // NS
EC = 80
EC2 = 2 * EC
NSC = EW // EC2
NCH_E = EW // EC
RC = 80
NCH_R = N // RC
RCH_PER_W = (NCH_R + NS - 1) // NS
NQ = H // L
SLOTS = 2 * K
DR = 640
DZW = DR // NS


def _bc16(v):
    return jnp.broadcast_to(jnp.asarray(v, jnp.int32), (L,))


def _iota16():
    return lax.iota(jnp.int32, L)


def _sc_body(xh2_ref, wf_ref, snd3_ref, rcv3_ref,
             tx_ref, sig_ref,
             acc_sh, deg_sh,
             ridx_all, dloc, rows0, rows1, rows2, rows3,
             sbuf0, sbuf1, sbuf2, sbuf3, wbuf0, wbuf1, wbuf2, wbuf3,
             gix0, gix1, gix2, gix3, zrow, idbuf, sgbuf,
             sem_g0, sem_g1, sem_g2, sem_g3, sem_s0, sem_s1, sem_s2, sem_s3,
             sem_f0, sem_f1, sem_f2, sem_f3, sem_d):
    ci = lax.axis_index("c")
    si = lax.axis_index("s")
    iot = _iota16()
    zeros = jnp.zeros((L,), jnp.float32)
    rows_b = (rows0, rows1, rows2, rows3)
    sbuf_b = (sbuf0, sbuf1, sbuf2, sbuf3)
    wbuf_b = (wbuf0, wbuf1, wbuf2, wbuf3)
    gix_b = (gix0, gix1, gix2, gix3)
    semg_b = (sem_g0, sem_g1, sem_g2, sem_g3)
    sems_b = (sem_s0, sem_s1, sem_s2, sem_s3)
    semf_b = (sem_f0, sem_f1, sem_f2, sem_f3)

    for i in range(RCH_PER_W):
        ch = si + i * NS
        base = ch * RC

        @pl.when(ch < NCH_R)
        def _():
            row0 = ci * N + base
            pltpu.sync_copy(xh2_ref.at[pl.ds(row0, RC)],
                            rows0.at[pl.ds(0, EC)])
            pltpu.sync_copy(rows0.at[pl.ds(0, EC)],
                            tx_ref.at[pl.ds(row0, RC)])
            pltpu.sync_copy(rows0.at[pl.ds(0, EC)],
                            acc_sh.at[pl.ds(base, RC)])

    pltpu.sync_copy(snd3_ref.at[si], ridx_all)

    def zero_dloc(i, c):
        plsc.store_scatter(dloc, [_bc16(i), iot], zeros)
        return c
    lax.fori_loop(0, DR, zero_dloc, 0)
    for r in range(DZW):
        plsc.store_scatter(zrow, [_bc16(r), iot], zeros)

    def fill_id(i, c):
        plsc.store_scatter(idbuf, [iot + i * L], iot + i * L)
        return c
    lax.fori_loop(0, DR // L, fill_id, 0)

    pltpu.sync_copy(zrow, deg_sh.at[pl.ds(si * DZW, DZW)])
    plsc.subcore_barrier()

    def deg_sc(jj, m2):
        pltpu.sync_copy(wf_ref.at[pl.ds(si * EW + jj * EC2, EC2)], wbuf0)
        for k in range(2):
            for g in range(EC // L):
                idx = ridx_all[2 * jj + k, pl.ds(g * L, L)]
                wv = wbuf0[pl.ds((k * (EC // L) + g) * L, L)]
                plsc.addupdate_scatter(
                    dloc,
                    [lax.shift_right_logical(idx, 4),
                     jnp.bitwise_and(idx, 15)],
                    wv)
                m2 = jnp.maximum(m2, -wv)
        return m2
    m2 = lax.fori_loop(0, NSC, deg_sc,
                       jnp.full((L,), -jnp.inf, jnp.float32))

    pltpu.sync_copy(dloc, deg_sh.at[idbuf], add=True)
    plsc.subcore_barrier()
    pltpu.sync_copy(deg_sh, dloc)

    def max_chunk(i, mv):
        return jnp.maximum(mv, dloc[i, pl.ds(0, L)])
    mv = lax.fori_loop(0, DR, max_chunk, m2)
    mmax = jnp.max(mv)
    gam2 = 0.5 * mmax * mmax
    gam3 = 0.25 * mmax * mmax

    @pl.when(jnp.logical_and(ci == 0, si == 0))
    def _():
        sigv = jnp.full((L,), 1.0, jnp.float32) / jnp.broadcast_to(mmax, (L,))
        plsc.store_scatter(sgbuf, [iot], sigv)
        pltpu.sync_copy(sgbuf, sig_ref)

    pltpu.sync_copy(rcv3_ref.at[si], ridx_all)

    def round_body(t, c):
        gam = jnp.where(t == 0, 0.0, jnp.where(t == 1, gam2, gam3))
        gamv = jnp.broadcast_to(gam, (L,))
        src_row = (2 * t + ci) * N
        prv_row = (2 * jnp.maximum(t - 1, 0) + ci) * N
        dst_row = (2 * (t + 1) + ci) * N

        def _init_fetch(i, b):
            ch = si + i * NS
            base = ch * RC

            @pl.when(ch < NCH_R)
            def _():
                pltpu.async_copy(tx_ref.at[pl.ds(prv_row + base, RC)],
                                 rows_b[b].at[pl.ds(EC, EC)], semf_b[b])

        def _init_do(i, b):
            ch = si + i * NS
            base = ch * RC
            oof = b * EC

            @pl.when(ch < NCH_R)
            def _():
                pltpu.sync_copy(acc_sh.at[pl.ds(base, RC)],
                                rows_b[b].at[pl.ds(0, EC)])
                pltpu.make_async_copy(tx_ref.at[pl.ds(0, RC)],
                                      rows_b[b].at[pl.ds(EC, EC)],
                                      semf_b[b]).wait()
                rw = rows_b[b]

                def grp_body(r5, c3):
                    dgrp = dloc[ch * (RC // L) + r5, pl.ds(0, L)]
                    for k16 in range(L):
                        r = r5 * L + k16
                        dspl = jnp.broadcast_to(dgrp[k16], (L,))
                        for q in range(NQ):
                            sl = pl.ds(q * L, L)
                            rows2[oof + r, sl] = (dspl * rw[r, sl]
                                                  - gamv * rw[EC + r, sl])
                    return c3
                lax.fori_loop(0, RC // L, grp_body, 0)
                pltpu.sync_copy(rows2.at[pl.ds(oof, EC)],
                                acc_sh.at[pl.ds(base, RC)])

        _init_fetch(0, 0)

        def init_pair(ii, c2):
            _init_fetch(2 * ii + 1, 1)
            _init_do(2 * ii, 0)

            @pl.when(ii < RCH_PER_W // 2 - 1)
            def _():
                _init_fetch(2 * ii + 2, 0)
            _init_do(2 * ii + 1, 1)
            return c2
        lax.fori_loop(0, RCH_PER_W // 2, init_pair, 0)
        plsc.subcore_barrier()

        def _build_gix(b, jj):
            for k in range(2):
                for g in range(EC // L):
                    sl = pl.ds(g * L, L)
                    gix_b[b][k, sl] = ridx_all[2 * jj + k, sl] + src_row

        def _start_gather(b):
            for k in range(2):
                pltpu.async_copy(tx_ref.at[gix_b[b].at[k]],
                                 rows_b[b].at[pl.ds(k * EC, EC)],
                                 semg_b[b])

        def _wait_gather(b):
            for k in range(2):
                pltpu.make_async_copy(
                    tx_ref.at[pl.ds(0, EC)],
                    rows_b[b].at[pl.ds(k * EC, EC)], semg_b[b]).wait()

        def _fetch_sw(b, jj):
            pltpu.async_copy(snd3_ref.at[si].at[pl.ds(2 * jj, 2)],
                             sbuf_b[b], semf_b[b])
            pltpu.async_copy(wf_ref.at[pl.ds(si * EW + jj * EC2, EC2)],
                             wbuf_b[b], semf_b[b])

        def _wait_sw(b):
            pltpu.make_async_copy(snd3_ref.at[si].at[pl.ds(0, 2)],
                                  sbuf_b[b], semf_b[b]).wait()
            pltpu.make_async_copy(wf_ref.at[pl.ds(0, EC2)],
                                  wbuf_b[b], semf_b[b]).wait()

        def _compute(b):
            rw = rows_b[b]
            wb = wbuf_b[b]

            def grp(g, c3):
                wg = -wb[pl.ds(g * L, L)]
                for k16 in range(L):
                    e = g * L + k16
                    w = jnp.broadcast_to(wg[k16], (L,))
                    for q in range(NQ):
                        sl = pl.ds(q * L, L)
                        rw[e, sl] = rw[e, sl] * w
                return c3
            lax.fori_loop(0, EC2 // L, grp, 0)

        def _start_scatter(b):
            for k in range(2):
                pltpu.async_copy(rows_b[b].at[pl.ds(k * EC, EC)],
                                 acc_sh.at[sbuf_b[b].at[k]],
                                 sems_b[b], add=True)

        def _wait_scatter(b):
            for k in range(2):
                pltpu.make_async_copy(
                    rows_b[b].at[pl.ds(k * EC, EC)],
                    acc_sh.at[pl.ds(0, EC)], sems_b[b]).wait()

        def _step(jj, b, wait_prev2, fetch_next):
            n1 = (b + 1) % 3
            if fetch_next:
                _build_gix(n1, jj + 1)
            if wait_prev2:
                _wait_scatter(n1)
            if fetch_next:
                _start_gather(n1)
                _fetch_sw(n1, jj + 1)
            _wait_gather(b)
            _wait_sw(b)
            _compute(b)
            _start_scatter(b)

        _fetch_sw(0, 0)
        _build_gix(0, 0)
        _start_gather(0)
        _step(0, 0, False, True)
        _step(1, 1, False, True)

        def tri_loop(p, c2):
            _step(3 * p + 2, 2, True, True)
            _step(3 * p + 3, 0, True, True)
            _step(3 * p + 4, 1, True, True)
            return c2
        lax.fori_loop(0, (NSC - 5) // 3, tri_loop, 0)
        _step(NSC - 3, 2, True, True)
        _step(NSC - 2, 0, True, True)
        _step(NSC - 1, 1, True, False)
        _wait_scatter(0)
        _wait_scatter(1)
        plsc.subcore_barrier()

        for i in range(RCH_PER_W):
            ch = si + i * NS
            base = ch * RC

            @pl.when(ch < NCH_R)
            def _():
                pltpu.async_copy(acc_sh.at[pl.ds(base, RC)],
                                 tx_ref.at[pl.ds(dst_row + base, RC)],
                                 sem_d)
        for i in range(RCH_PER_W):
            ch = si + i * NS
            base = ch * RC

            @pl.when(ch < NCH_R)
            def _():
                pltpu.make_async_copy(
                    acc_sh.at[pl.ds(base, RC)],
                    tx_ref.at[pl.ds(dst_row + base, RC)],
                    sem_d).wait()
        plsc.subcore_barrier()
        return c

    lax.fori_loop(0, K - 1, round_body, 0)


@jax.jit
def _sc_cheb(xh2, wf, snd3, rcv3):
    mesh = plsc.VectorSubcoreMesh(core_axis_name="c", subcore_axis_name="s",
                                  num_cores=NC, num_subcores=NS)
    return pl.kernel(
        _sc_body,
        out_type=(
            jax.ShapeDtypeStruct((SLOTS * N, H), jnp.float32),
            jax.ShapeDtypeStruct((L,), jnp.float32),
        ),
        mesh=mesh,
        compiler_params=pltpu.CompilerParams(needs_layout_passes=False,
                                             use_tc_tiling_on_sc=False),
        scratch_types=[
            pltpu.VMEM_SHARED((N, H), jnp.float32),
            pltpu.VMEM_SHARED((DR, L), jnp.float32),
            pltpu.VMEM((NCH_E, EC), jnp.int32),
            pltpu.VMEM((DR, L), jnp.float32),
            pltpu.VMEM((EC2, H), jnp.float32),
            pltpu.VMEM((EC2, H), jnp.float32),
            pltpu.VMEM((EC2, H), jnp.float32),
            pltpu.VMEM((EC2, H), jnp.float32),
            pltpu.VMEM((2, EC), jnp.int32),
            pltpu.VMEM((2, EC), jnp.int32),
            pltpu.VMEM((2, EC), jnp.int32),
            pltpu.VMEM((2, EC), jnp.int32),
            pltpu.VMEM((EC2,), jnp.float32),
            pltpu.VMEM((EC2,), jnp.float32),
            pltpu.VMEM((EC2,), jnp.float32),
            pltpu.VMEM((EC2,), jnp.float32),
            pltpu.VMEM((2, EC), jnp.int32),
            pltpu.VMEM((2, EC), jnp.int32),
            pltpu.VMEM((2, EC), jnp.int32),
            pltpu.VMEM((2, EC), jnp.int32),
            pltpu.VMEM((DZW, L), jnp.float32),
            pltpu.VMEM((DR,), jnp.int32),
            pltpu.VMEM((L,), jnp.float32),
            pltpu.SemaphoreType.DMA,
            pltpu.SemaphoreType.DMA,
            pltpu.SemaphoreType.DMA,
            pltpu.SemaphoreType.DMA,
            pltpu.SemaphoreType.DMA,
            pltpu.SemaphoreType.DMA,
            pltpu.SemaphoreType.DMA,
            pltpu.SemaphoreType.DMA,
            pltpu.SemaphoreType.DMA,
            pltpu.SemaphoreType.DMA,
            pltpu.SemaphoreType.DMA,
            pltpu.SemaphoreType.DMA,
            pltpu.SemaphoreType.DMA,
        ],
    )(xh2, wf, snd3, rcv3)


RB = 1000
NRB = N // RB


def _tc_body(sig_ref, tx_ref, w_ref, bsum_ref, out_ref):
    s = pl.program_id(1)
    k = s // 2
    sg = sig_ref[0]
    sk = jnp.float32(1.0)
    p = jnp.float32(1.0)
    for kk in range(1, K):
        p = p * sg if kk == 1 else p * (2.0 * sg)
        sk = jnp.where(k == kk, p, sk)
    contrib = jnp.dot(tx_ref[...], w_ref[0],
                      preferred_element_type=jnp.float32) * sk

    @pl.when(s == 0)
    def _():
        out_ref[...] = contrib + bsum_ref[...]

    @pl.when(s > 0)
    def _():
        out_ref[...] = out_ref[...] + contrib


@jax.jit
def _tc_combine(tx, sig, w3, bsum):
    grid = (NRB, SLOTS)
    return pl.pallas_call(
        _tc_body,
        grid=grid,
        in_specs=[
            pl.BlockSpec(memory_space=pltpu.SMEM),
            pl.BlockSpec((RB, H), lambda i, s: (s * NRB + i, 0)),
            pl.BlockSpec((1, H, OUT), lambda i, s: (s, 0, 0)),
            pl.BlockSpec((1, OUT), lambda i, s: (0, 0)),
        ],
        out_specs=pl.BlockSpec((RB, OUT), lambda i, s: (i, 0)),
        out_shape=jax.ShapeDtypeStruct((N, OUT), jnp.float32),
    )(sig, tx, w3, bsum)


def kernel(x, edge_weight, W, b, bias, senders, receivers):
    xh2 = x.reshape(N, NC, H).transpose(1, 0, 2).reshape(NC * N, H)
    wf = edge_weight
    snd3 = senders.astype(jnp.int32).reshape(NS, NCH_E, EC)
    rcv3 = receivers.astype(jnp.int32).reshape(NS, NCH_E, EC)
    tx, sig = _sc_cheb(xh2, wf, snd3, rcv3)
    w3d = W.reshape(K, NC, H, OUT).reshape(SLOTS, H, OUT)
    bsum = (b.sum(axis=0) + bias).reshape(1, OUT)
    return _tc_combine(tx, sig, w3d, bsum)


# speedup vs baseline: 9.5225x; 1.0889x over previous
import functools

import jax
import jax.numpy as jnp
from jax import lax
from jax.experimental import pallas as pl
from jax.experimental.pallas import tpu as pltpu
from jax.experimental.pallas import tpu_sc as plsc

N = 10000
E = 320000
D = 128
K = 6
OUT = 128

NC = 2
NS = 16
L = 16

H = D // NC
EW = E // NS
EC = 80
EC2 = 2 * EC
NSC = EW // EC2
NCH_E = EW // EC
RC = 80
NCH_R = N // RC
RCH_PER_W = (NCH_R + NS - 1) // NS
NQ = H // L
SLOTS = 2 * K
DR = 640
DZW = DR // NS


def _bc16(v):
    return jnp.broadcast_to(jnp.asarray(v, jnp.int32), (L,))


def _iota16():
    return lax.iota(jnp.int32, L)


def _sc_body(xh2_ref, wf_ref, snd3_ref, rcv3_ref,
             tx_ref, sig_ref,
             acc_sh, deg_sh,
             ridx_all, dloc, rows0, rows1, rows2, rows3,
             sbuf0, sbuf1, sbuf2, sbuf3, wbuf0, wbuf1, wbuf2, wbuf3,
             gix0, gix1, gix2, gix3, zrow, idbuf, sgbuf,
             sem_g0, sem_g1, sem_g2, sem_g3, sem_s0, sem_s1, sem_s2, sem_s3,
             sem_f0, sem_f1, sem_f2, sem_f3, sem_d):
    ci = lax.axis_index("c")
    si = lax.axis_index("s")
    iot = _iota16()
    zeros = jnp.zeros((L,), jnp.float32)
    rows_b = (rows0, rows1, rows2, rows3)
    sbuf_b = (sbuf0, sbuf1, sbuf2, sbuf3)
    wbuf_b = (wbuf0, wbuf1, wbuf2, wbuf3)
    gix_b = (gix0, gix1, gix2, gix3)
    semg_b = (sem_g0, sem_g1, sem_g2, sem_g3)
    sems_b = (sem_s0, sem_s1, sem_s2, sem_s3)
    semf_b = (sem_f0, sem_f1, sem_f2, sem_f3)

    for i in range(RCH_PER_W):
        ch = si + i * NS
        base = ch * RC

        @pl.when(ch < NCH_R)
        def _():
            row0 = ci * N + base
            pltpu.sync_copy(xh2_ref.at[pl.ds(row0, RC)],
                            rows0.at[pl.ds(0, EC)])
            pltpu.sync_copy(rows0.at[pl.ds(0, EC)],
                            tx_ref.at[pl.ds(row0, RC)])
            pltpu.sync_copy(rows0.at[pl.ds(0, EC)],
                            acc_sh.at[pl.ds(base, RC)])

    pltpu.sync_copy(snd3_ref.at[si], ridx_all)

    def zero_dloc(i, c):
        plsc.store_scatter(dloc, [_bc16(i), iot], zeros)
        return c
    lax.fori_loop(0, DR, zero_dloc, 0)
    for r in range(DZW):
        plsc.store_scatter(zrow, [_bc16(r), iot], zeros)

    def fill_id(i, c):
        plsc.store_scatter(idbuf, [iot + i * L], iot + i * L)
        return c
    lax.fori_loop(0, DR // L, fill_id, 0)

    pltpu.sync_copy(zrow, deg_sh.at[pl.ds(si * DZW, DZW)])
    plsc.subcore_barrier()

    def deg_sc(jj, m2):
        pltpu.sync_copy(wf_ref.at[pl.ds(si * EW + jj * EC2, EC2)], wbuf0)
        for k in range(2):
            for g in range(EC // L):
                idx = ridx_all[2 * jj + k, pl.ds(g * L, L)]
                wv = wbuf0[pl.ds((k * (EC // L) + g) * L, L)]
                plsc.addupdate_scatter(
                    dloc,
                    [lax.shift_right_logical(idx, 4),
                     jnp.bitwise_and(idx, 15)],
                    wv)
                m2 = jnp.maximum(m2, -wv)
        return m2
    m2 = lax.fori_loop(0, NSC, deg_sc,
                       jnp.full((L,), -jnp.inf, jnp.float32))

    pltpu.sync_copy(dloc, deg_sh.at[idbuf], add=True)
    plsc.subcore_barrier()
    pltpu.sync_copy(deg_sh, dloc)

    def max_chunk(i, mv):
        return jnp.maximum(mv, dloc[i, pl.ds(0, L)])
    mv = lax.fori_loop(0, DR, max_chunk, m2)
    mmax = jnp.max(mv)
    gam2 = 0.5 * mmax * mmax
    gam3 = 0.25 * mmax * mmax

    @pl.when(jnp.logical_and(ci == 0, si == 0))
    def _():
        sigv = jnp.full((L,), 1.0, jnp.float32) / jnp.broadcast_to(mmax, (L,))
        plsc.store_scatter(sgbuf, [iot], sigv)
        pltpu.sync_copy(sgbuf, sig_ref)

    pltpu.sync_copy(rcv3_ref.at[si], ridx_all)

    def round_body(t, c):
        gam = jnp.where(t == 0, 0.0, jnp.where(t == 1, gam2, gam3))
        gamv = jnp.broadcast_to(gam, (L,))
        src_row = (2 * t + ci) * N
        prv_row = (2 * jnp.maximum(t - 1, 0) + ci) * N
        dst_row = (2 * (t + 1) + ci) * N

        def _init_fetch(i, b):
            ch = si + i * NS
            base = ch * RC

            @pl.when(ch < NCH_R)
            def _():
                pltpu.async_copy(tx_ref.at[pl.ds(prv_row + base, RC)],
                                 rows_b[b].at[pl.ds(EC, EC)], semf_b[b])

        def _init_do(i, b):
            ch = si + i * NS
            base = ch * RC
            oof = b * EC

            @pl.when(ch < NCH_R)
            def _():
                pltpu.sync_copy(acc_sh.at[pl.ds(base, RC)],
                                rows_b[b].at[pl.ds(0, EC)])
                pltpu.make_async_copy(tx_ref.at[pl.ds(0, RC)],
                                      rows_b[b].at[pl.ds(EC, EC)],
                                      semf_b[b]).wait()
                rw = rows_b[b]

                def grp_body(r5, c3):
                    dgrp = dloc[ch * (RC // L) + r5, pl.ds(0, L)]
                    for k16 in range(L):
                        r = r5 * L + k16
                        dspl = jnp.broadcast_to(dgrp[k16], (L,))
                        for q in range(NQ):
                            sl = pl.ds(q * L, L)
                            rows2[oof + r, sl] = (dspl * rw[r, sl]
                                                  - gamv * rw[EC + r, sl])
                    return c3
                lax.fori_loop(0, RC // L, grp_body, 0)
                pltpu.sync_copy(rows2.at[pl.ds(oof, EC)],
                                acc_sh.at[pl.ds(base, RC)])

        _init_fetch(0, 0)

        def init_pair(ii, c2):
            _init_fetch(2 * ii + 1, 1)
            _init_do(2 * ii, 0)

            @pl.when(ii < RCH_PER_W // 2 - 1)
            def _():
                _init_fetch(2 * ii + 2, 0)
            _init_do(2 * ii + 1, 1)
            return c2
        lax.fori_loop(0, RCH_PER_W // 2, init_pair, 0)
        plsc.subcore_barrier()

        def _build_gix(b, jj):
            for k in range(2):
                for g in range(EC // L):
                    sl = pl.ds(g * L, L)
                    gix_b[b][k, sl] = ridx_all[2 * jj + k, sl] + src_row

        def _start_gather(b):
            for k in range(2):
                pltpu.async_copy(tx_ref.at[gix_b[b].at[k]],
                                 rows_b[b].at[pl.ds(k * EC, EC)],
                                 semg_b[b])

        def _wait_gather(b):
            for k in range(2):
                pltpu.make_async_copy(
                    tx_ref.at[pl.ds(0, EC)],
                    rows_b[b].at[pl.ds(k * EC, EC)], semg_b[b]).wait()

        def _fetch_sw(b, jj):
            pltpu.async_copy(snd3_ref.at[si].at[pl.ds(2 * jj, 2)],
                             sbuf_b[b], semf_b[b])
            pltpu.async_copy(wf_ref.at[pl.ds(si * EW + jj * EC2, EC2)],
                             wbuf_b[b], semf_b[b])

        def _wait_sw(b):
            pltpu.make_async_copy(snd3_ref.at[si].at[pl.ds(0, 2)],
                                  sbuf_b[b], semf_b[b]).wait()
            pltpu.make_async_copy(wf_ref.at[pl.ds(0, EC2)],
                                  wbuf_b[b], semf_b[b]).wait()

        def _compute(b):
            rw = rows_b[b]
            wb = wbuf_b[b]

            def grp(g, c3):
                wg = -wb[pl.ds(g * L, L)]
                for k16 in range(L):
                    e = g * L + k16
                    w = jnp.broadcast_to(wg[k16], (L,))
                    for q in range(NQ):
                        sl = pl.ds(q * L, L)
                        rw[e, sl] = rw[e, sl] * w
                return c3
            lax.fori_loop(0, EC2 // L, grp, 0)

        def _start_scatter(b):
            for k in range(2):
                pltpu.async_copy(rows_b[b].at[pl.ds(k * EC, EC)],
                                 acc_sh.at[sbuf_b[b].at[k]],
                                 sems_b[b], add=True)

        def _wait_scatter(b):
            for k in range(2):
                pltpu.make_async_copy(
                    rows_b[b].at[pl.ds(k * EC, EC)],
                    acc_sh.at[pl.ds(0, EC)], sems_b[b]).wait()

        def _prep(jj2, b2):
            _build_gix(b2, jj2)
            _start_gather(b2)
            _fetch_sw(b2, jj2)

        def _step(jj, b, wait_prev2, prep_next):
            n2 = (b + 2) % 4
            if wait_prev2:
                _wait_scatter(n2)
            if prep_next:
                _prep(jj + 2, n2)
            _wait_gather(b)
            _wait_sw(b)
            _compute(b)
            _start_scatter(b)

        _prep(0, 0)
        _prep(1, 1)
        _step(0, 0, False, True)
        _step(1, 1, False, True)

        def quad_loop(p, c2):
            _step(4 * p + 2, 2, True, True)
            _step(4 * p + 3, 3, True, True)
            _step(4 * p + 4, 0, True, True)
            _step(4 * p + 5, 1, True, True)
            return c2
        lax.fori_loop(0, (NSC - 5) // 4, quad_loop, 0)
        _step(NSC - 3, 2, True, True)
        _step(NSC - 2, 3, True, False)
        _step(NSC - 1, 0, True, False)
        _wait_scatter(3)
        _wait_scatter(0)
        plsc.subcore_barrier()

        for i in range(RCH_PER_W):
            ch = si + i * NS
            base = ch * RC

            @pl.when(ch < NCH_R)
            def _():
                pltpu.async_copy(acc_sh.at[pl.ds(base, RC)],
                                 tx_ref.at[pl.ds(dst_row + base, RC)],
                                 sem_d)
        for i in range(RCH_PER_W):
            ch = si + i * NS
            base = ch * RC

            @pl.when(ch < NCH_R)
            def _():
                pltpu.make_async_copy(
                    acc_sh.at[pl.ds(base, RC)],
                    tx_ref.at[pl.ds(dst_row + base, RC)],
                    sem_d).wait()
        plsc.subcore_barrier()
        return c

    lax.fori_loop(0, K - 1, round_body, 0)


@jax.jit
def _sc_cheb(xh2, wf, snd3, rcv3):
    mesh = plsc.VectorSubcoreMesh(core_axis_name="c", subcore_axis_name="s",
                                  num_cores=NC, num_subcores=NS)
    return pl.kernel(
        _sc_body,
        out_type=(
            jax.ShapeDtypeStruct((SLOTS * N, H), jnp.float32),
            jax.ShapeDtypeStruct((L,), jnp.float32),
        ),
        mesh=mesh,
        compiler_params=pltpu.CompilerParams(needs_layout_passes=False,
                                             use_tc_tiling_on_sc=False),
        scratch_types=[
            pltpu.VMEM_SHARED((N, H), jnp.float32),
            pltpu.VMEM_SHARED((DR, L), jnp.float32),
            pltpu.VMEM((NCH_E, EC), jnp.int32),
            pltpu.VMEM((DR, L), jnp.float32),
            pltpu.VMEM((EC2, H), jnp.float32),
            pltpu.VMEM((EC2, H), jnp.float32),
            pltpu.VMEM((EC2, H), jnp.float32),
            pltpu.VMEM((EC2, H), jnp.float32),
            pltpu.VMEM((2, EC), jnp.int32),
            pltpu.VMEM((2, EC), jnp.int32),
            pltpu.VMEM((2, EC), jnp.int32),
            pltpu.VMEM((2, EC), jnp.int32),
            pltpu.VMEM((EC2,), jnp.float32),
            pltpu.VMEM((EC2,), jnp.float32),
            pltpu.VMEM((EC2,), jnp.float32),
            pltpu.VMEM((EC2,), jnp.float32),
            pltpu.VMEM((2, EC), jnp.int32),
            pltpu.VMEM((2, EC), jnp.int32),
            pltpu.VMEM((2, EC), jnp.int32),
            pltpu.VMEM((2, EC), jnp.int32),
            pltpu.VMEM((DZW, L), jnp.float32),
            pltpu.VMEM((DR,), jnp.int32),
            pltpu.VMEM((L,), jnp.float32),
            pltpu.SemaphoreType.DMA,
            pltpu.SemaphoreType.DMA,
            pltpu.SemaphoreType.DMA,
            pltpu.SemaphoreType.DMA,
            pltpu.SemaphoreType.DMA,
            pltpu.SemaphoreType.DMA,
            pltpu.SemaphoreType.DMA,
            pltpu.SemaphoreType.DMA,
            pltpu.SemaphoreType.DMA,
            pltpu.SemaphoreType.DMA,
            pltpu.SemaphoreType.DMA,
            pltpu.SemaphoreType.DMA,
            pltpu.SemaphoreType.DMA,
        ],
    )(xh2, wf, snd3, rcv3)


RB = 1000
NRB = N // RB


def _tc_body(sig_ref, tx_ref, w_ref, bsum_ref, out_ref):
    s = pl.program_id(1)
    k = s // 2
    sg = sig_ref[0]
    sk = jnp.float32(1.0)
    p = jnp.float32(1.0)
    for kk in range(1, K):
        p = p * sg if kk == 1 else p * (2.0 * sg)
        sk = jnp.where(k == kk, p, sk)
    contrib = jnp.dot(tx_ref[...], w_ref[0],
                      preferred_element_type=jnp.float32) * sk

    @pl.when(s == 0)
    def _():
        out_ref[...] = contrib + bsum_ref[...]

    @pl.when(s > 0)
    def _():
        out_ref[...] = out_ref[...] + contrib


@jax.jit
def _tc_combine(tx, sig, w3, bsum):
    grid = (NRB, SLOTS)
    return pl.pallas_call(
        _tc_body,
        grid=grid,
        in_specs=[
            pl.BlockSpec(memory_space=pltpu.SMEM),
            pl.BlockSpec((RB, H), lambda i, s: (s * NRB + i, 0)),
            pl.BlockSpec((1, H, OUT), lambda i, s: (s, 0, 0)),
            pl.BlockSpec((1, OUT), lambda i, s: (0, 0)),
        ],
        out_specs=pl.BlockSpec((RB, OUT), lambda i, s: (i, 0)),
        out_shape=jax.ShapeDtypeStruct((N, OUT), jnp.float32),
    )(sig, tx, w3, bsum)


def kernel(x, edge_weight, W, b, bias, senders, receivers):
    xh2 = x.reshape(N, NC, H).transpose(1, 0, 2).reshape(NC * N, H)
    wf = edge_weight
    snd3 = senders.astype(jnp.int32).reshape(NS, NCH_E, EC)
    rcv3 = receivers.astype(jnp.int32).reshape(NS, NCH_E, EC)
    tx, sig = _sc_cheb(xh2, wf, snd3, rcv3)
    w3d = W.reshape(K, NC, H, OUT).reshape(SLOTS, H, OUT)
    bsum = (b.sum(axis=0) + bias).reshape(1, OUT)
    return _tc_combine(tx, sig, w3d, bsum)
